# Initial kernel scaffold; baseline (speedup 1.0000x reference)
import functools

import jax
import jax.numpy as jnp
from jax import lax
from jax.experimental import pallas as pl
from jax.experimental.pallas import tpu as pltpu
from jax.experimental.pallas import tpu_sc as plsc

F32 = jnp.float32

N_NODES = 100000
N_EDGES = 1600000
F_IN, F_H, F_OUT = 11, 64, 1

NC, NS = 2, 16
CW = 16
CHUNKS = F_H // CW

NP = 100352
EP = 1605632
ROWS = EP // 128
TILE_N = NP // NS
GB = 4
PROP_ROWS = ROWS // NS
PROP_GROUPS = PROP_ROWS // GB
DEG_ROWS = ROWS // (NC * NS)
DEG_GROUPS = DEG_ROWS // GB

BN = 2048
GRID = NP // BN

_MESH = plsc.VectorSubcoreMesh(core_axis_name="c", subcore_axis_name="s")


def _deg_body(dst_hbm, deg0, deg1, dstv, ones_v, zeros1, acc1):
    c = lax.axis_index("c")
    s = lax.axis_index("s")

    def _ones(i, _):
        ones_v[pl.ds(i * 16, 16)] = jnp.full((16,), 1.0, F32)
        return 0

    lax.fori_loop(0, 8, _ones, 0)

    def _zero(i, _):
        zeros1[pl.ds(i * 16, 16)] = jnp.zeros((16,), F32)
        return 0

    lax.fori_loop(0, TILE_N // 16, _zero, 0)
    sl = pl.ds(s * TILE_N, TILE_N)
    pltpu.sync_copy(zeros1, acc1.at[sl])
    plsc.subcore_barrier()

    row0 = (c * NS + s) * DEG_ROWS

    def _group(i, _):
        pltpu.sync_copy(dst_hbm.at[pl.ds(row0 + i * GB, GB)], dstv)

        def _scat(j, _):
            pltpu.sync_copy(ones_v, acc1.at[dstv.at[j]], add=True)
            return 0

        lax.fori_loop(0, GB, _scat, 0)
        return 0

    lax.fori_loop(0, DEG_GROUPS, _group, 0)
    plsc.subcore_barrier()

    @pl.when(c == 0)
    def _():
        pltpu.sync_copy(acc1.at[sl], deg0.at[sl])

    @pl.when(c == 1)
    def _():
        pltpu.sync_copy(acc1.at[sl], deg1.at[sl])


_deg_call = pl.kernel(
    _deg_body,
    out_type=(
        jax.ShapeDtypeStruct((NP,), F32),
        jax.ShapeDtypeStruct((NP,), F32),
    ),
    mesh=_MESH,
    scratch_types=[
        pltpu.VMEM((GB, 128), jnp.int32),
        pltpu.VMEM((128,), F32),
        pltpu.VMEM((TILE_N,), F32),
        pltpu.VMEM_SHARED((NP,), F32),
    ],
)


def _prop_body(hp0, hp1, hp2, hp3, src_hbm, dst_hbm,
               o0, o1, o2, o3,
               srcv0, srcv1, dstv0, dstv1, rows0, rows1,
               zeros_v, acc, sem0, sem1):
    c = lax.axis_index("c")
    s = lax.axis_index("s")
    hps = (hp0, hp1, hp2, hp3)
    outs = (o0, o1, o2, o3)
    srcv = (srcv0, srcv1)
    dstv = (dstv0, dstv1)
    rows = (rows0, rows1)
    sems = (sem0, sem1)

    def _zinit(i, _):
        zeros_v[i, :] = jnp.zeros((16,), F32)
        return 0

    lax.fori_loop(0, zeros_v.shape[0], _zinit, 0)

    nbase = s * TILE_N
    row0 = s * PROP_ROWS

    def _do_pass(hp, out):
        def _zcopy(i, _):
            pltpu.sync_copy(
                zeros_v, acc.at[pl.ds(nbase + i * zeros_v.shape[0],
                                      zeros_v.shape[0])])
            return 0

        lax.fori_loop(0, TILE_N // zeros_v.shape[0], _zcopy, 0)
        plsc.subcore_barrier()

        def _stage_fire(g, b):
            r = row0 + g * GB
            pltpu.sync_copy(src_hbm.at[pl.ds(r, GB)], srcv[b])
            pltpu.sync_copy(dst_hbm.at[pl.ds(r, GB)], dstv[b])

            def _fire(j, _):
                pltpu.async_copy(hp.at[srcv[b].at[j]],
                                 rows[b].at[pl.ds(j * 128, 128)], sems[b])
                return 0

            lax.fori_loop(0, GB, _fire, 0)

        _stage_fire(0, 0)
        _stage_fire(1, 1)

        def _outer(i, _):
            g = i * 2
            for b in (0, 1):
                gg = g + b
                pltpu.make_async_copy(
                    hp.at[pl.ds(0, GB * 128)], rows[b], sems[b]).wait()

                def _scat(j, _):
                    pltpu.sync_copy(rows[b].at[pl.ds(j * 128, 128)],
                                    acc.at[dstv[b].at[j]], add=True)
                    return 0

                lax.fori_loop(0, GB, _scat, 0)

                @pl.when(gg + 2 < PROP_GROUPS)
                def _():
                    _stage_fire(gg + 2, b)

            return 0

        lax.fori_loop(0, PROP_GROUPS // 2, _outer, 0)
        plsc.subcore_barrier()
        pltpu.sync_copy(acc.at[pl.ds(nbase, TILE_N)],
                        out.at[pl.ds(nbase, TILE_N)])
        plsc.subcore_barrier()

    for p in range(2):
        for half in range(2):
            q = half * 2 + p

            @pl.when(c == half)
            def _(q=q):
                _do_pass(hps[q], outs[q])


_prop_call = pl.kernel(
    _prop_body,
    out_type=tuple(jax.ShapeDtypeStruct((NP, CW), F32) for _ in range(CHUNKS)),
    mesh=_MESH,
    scratch_types=[
        pltpu.VMEM((GB, 128), jnp.int32),
        pltpu.VMEM((GB, 128), jnp.int32),
        pltpu.VMEM((GB, 128), jnp.int32),
        pltpu.VMEM((GB, 128), jnp.int32),
        pltpu.VMEM((GB * 128, CW), F32),
        pltpu.VMEM((GB * 128, CW), F32),
        pltpu.VMEM((784, CW), F32),
        pltpu.VMEM_SHARED((NP, CW), F32),
        pltpu.SemaphoreType.DMA,
        pltpu.SemaphoreType.DMA,
    ],
)


def _pre_body(x_ref, d0_ref, d1_ref, w1_ref, dinv_ref, *hp_refs):
    deg = d0_ref[...] + d1_ref[...] + 1.0
    dinv = lax.rsqrt(deg)
    dinv_ref[...] = dinv
    h = jnp.dot(x_ref[...], w1_ref[...], preferred_element_type=F32)
    for q in range(CHUNKS):
        hp_refs[q][...] = dinv * h[:, q * CW:(q + 1) * CW]


def _mid_body(a0, a1, a2, a3, h0, h1, h2, h3, dinv_ref, b1_ref, w2_ref,
              *out_refs):
    aggs = (a0, a1, a2, a3)
    hps = (h0, h1, h2, h3)
    dinv = dinv_ref[...]
    b = b1_ref[...]
    z = jnp.concatenate(
        [jax.nn.relu(dinv * (aggs[q][...] + hps[q][...])
                     + b[:, q * CW:(q + 1) * CW])
         for q in range(CHUNKS)], axis=1)
    h = jnp.dot(z, w2_ref[...], preferred_element_type=F32)
    for q in range(CHUNKS):
        out_refs[q][...] = dinv * h[:, q * CW:(q + 1) * CW]


def _post_body(a0, a1, a2, a3, h0, h1, h2, h3, dinv_ref, b2_ref, wl_ref,
               bl_ref, out_ref):
    aggs = (a0, a1, a2, a3)
    hps = (h0, h1, h2, h3)
    dinv = dinv_ref[...]
    b = b2_ref[...]
    z = jnp.concatenate(
        [jax.nn.relu(dinv * (aggs[q][...] + hps[q][...])
                     + b[:, q * CW:(q + 1) * CW])
         for q in range(CHUNKS)], axis=1)
    out_ref[...] = (jnp.dot(z, wl_ref[...], preferred_element_type=F32)
                    + bl_ref[...])


def _row_spec(w):
    return pl.BlockSpec((BN, w), lambda i: (i, 0))


def _full_spec(r, cdim):
    return pl.BlockSpec((r, cdim), lambda i: (0, 0))


_pre_call = pl.pallas_call(
    _pre_body,
    grid=(GRID,),
    in_specs=[_row_spec(F_IN), _row_spec(1), _row_spec(1),
              _full_spec(F_IN, F_H)],
    out_specs=[_row_spec(1)] + [_row_spec(CW)] * CHUNKS,
    out_shape=[jax.ShapeDtypeStruct((NP, 1), F32)]
    + [jax.ShapeDtypeStruct((NP, CW), F32) for _ in range(CHUNKS)],
)

_mid_call = pl.pallas_call(
    _mid_body,
    grid=(GRID,),
    in_specs=[_row_spec(CW)] * (2 * CHUNKS)
    + [_row_spec(1), _full_spec(1, F_H), _full_spec(F_H, F_H)],
    out_specs=[_row_spec(CW)] * CHUNKS,
    out_shape=[jax.ShapeDtypeStruct((NP, CW), F32) for _ in range(CHUNKS)],
)

_post_call = pl.pallas_call(
    _post_body,
    grid=(GRID,),
    in_specs=[_row_spec(CW)] * (2 * CHUNKS)
    + [_row_spec(1), _full_spec(1, F_H), _full_spec(F_H, F_OUT),
       _full_spec(1, F_OUT)],
    out_specs=_row_spec(F_OUT),
    out_shape=jax.ShapeDtypeStruct((NP, F_OUT), F32),
)


def kernel(x, edge_index, W1, b1, W2, b2, Wl, bl):
    src = edge_index[0]
    dst = edge_index[1]
    n_pad_rows = NP - N_NODES
    pad_idx = N_NODES + (jnp.arange(EP - N_EDGES, dtype=jnp.int32)
                         % n_pad_rows)
    src_p = jnp.concatenate([src, pad_idx]).reshape(ROWS, 128)
    dst_p = jnp.concatenate([dst, pad_idx]).reshape(ROWS, 128)
    x_p = jnp.pad(x, ((0, NP - N_NODES), (0, 0)))

    deg0, deg1 = _deg_call(dst_p)
    dinv, *h1p = _pre_call(x_p, deg0.reshape(NP, 1), deg1.reshape(NP, 1), W1)
    agg1 = _prop_call(*h1p, src_p, dst_p)
    h2p = _mid_call(*agg1, *h1p, dinv, b1.reshape(1, F_H), W2)
    agg2 = _prop_call(*h2p, src_p, dst_p)
    out = _post_call(*agg2, *h2p, dinv, b2.reshape(1, F_H), Wl,
                     bl.reshape(1, F_OUT))
    return out[:N_NODES]


# speedup vs baseline: 16.6526x; 16.6526x over previous
import functools

import jax
import jax.numpy as jnp
from jax import lax
from jax.experimental import pallas as pl
from jax.experimental.pallas import tpu as pltpu
from jax.experimental.pallas import tpu_sc as plsc

F32 = jnp.float32

N_NODES = 100000
N_EDGES = 1600000
F_IN, F_H, F_OUT = 11, 64, 1

NC, NS = 2, 16
CW = 16
CHUNKS = F_H // CW

NP = 100096
EP = 1605632
ROWS = EP // 128
TILE_N = NP // NS
GB = 4
PROP_ROWS = ROWS // NS
PROP_GROUPS = PROP_ROWS // GB
DEG_ROWS = ROWS // (NC * NS)
DEG_GROUPS = DEG_ROWS // GB

BN = 3128
GRID = NP // BN

_MESH = plsc.VectorSubcoreMesh(core_axis_name="c", subcore_axis_name="s")
_SC_PARAMS = pltpu.CompilerParams(use_tc_tiling_on_sc=False)


def _deg_body(dst_hbm, deg0, deg1, dstv, ones_v, zeros1, acc1):
    c = lax.axis_index("c")
    s = lax.axis_index("s")

    def _ones(i, _):
        ones_v[pl.ds(i * 16, 16)] = jnp.full((16,), 1.0, F32)
        return 0

    lax.fori_loop(0, 8, _ones, 0)

    def _zero(i, _):
        zeros1[pl.ds(i * 16, 16)] = jnp.zeros((16,), F32)
        return 0

    lax.fori_loop(0, TILE_N // 16, _zero, 0)
    sl = pl.ds(s * TILE_N, TILE_N)
    pltpu.sync_copy(zeros1, acc1.at[sl])
    plsc.subcore_barrier()

    row0 = (c * NS + s) * DEG_ROWS

    def _group(i, _):
        pltpu.sync_copy(dst_hbm.at[pl.ds(row0 + i * GB, GB)], dstv)

        def _scat(j, _):
            pltpu.sync_copy(ones_v, acc1.at[dstv.at[j]], add=True)
            return 0

        lax.fori_loop(0, GB, _scat, 0)
        return 0

    lax.fori_loop(0, DEG_GROUPS, _group, 0)
    plsc.subcore_barrier()

    @pl.when(c == 0)
    def _():
        pltpu.sync_copy(acc1.at[sl], deg0.at[sl])

    @pl.when(c == 1)
    def _():
        pltpu.sync_copy(acc1.at[sl], deg1.at[sl])


_deg_call = pl.kernel(
    _deg_body,
    out_type=(
        jax.ShapeDtypeStruct((NP,), F32),
        jax.ShapeDtypeStruct((NP,), F32),
    ),
    mesh=_MESH,
    scratch_types=[
        pltpu.VMEM((GB, 128), jnp.int32),
        pltpu.VMEM((128,), F32),
        pltpu.VMEM((TILE_N,), F32),
        pltpu.VMEM_SHARED((NP,), F32),
    ],
    compiler_params=_SC_PARAMS,
)


def _prop_body(hp0, hp1, hp2, hp3, src_hbm, dst_hbm,
               o0, o1, o2, o3,
               srcv0, srcv1, dstv0, dstv1, rows0, rows1,
               zeros_v, acc, sem0, sem1):
    c = lax.axis_index("c")
    s = lax.axis_index("s")
    hps = (hp0, hp1, hp2, hp3)
    outs = (o0, o1, o2, o3)
    srcv = (srcv0, srcv1)
    dstv = (dstv0, dstv1)
    rows = (rows0, rows1)
    sems = (sem0, sem1)

    def _zinit(i, _):
        zeros_v[i, :] = jnp.zeros((16,), F32)
        return 0

    lax.fori_loop(0, zeros_v.shape[0], _zinit, 0)

    nbase = s * TILE_N
    row0 = s * PROP_ROWS

    def _do_pass(hp, out):
        def _zcopy(i, _):
            pltpu.sync_copy(
                zeros_v, acc.at[pl.ds(nbase + i * zeros_v.shape[0],
                                      zeros_v.shape[0])])
            return 0

        lax.fori_loop(0, TILE_N // zeros_v.shape[0], _zcopy, 0)
        plsc.subcore_barrier()

        def _stage_fire(g, b):
            r = row0 + g * GB
            pltpu.sync_copy(src_hbm.at[pl.ds(r, GB)], srcv[b])
            pltpu.sync_copy(dst_hbm.at[pl.ds(r, GB)], dstv[b])

            def _fire(j, _):
                pltpu.async_copy(hp.at[srcv[b].at[j]],
                                 rows[b].at[pl.ds(j * 128, 128)], sems[b])
                return 0

            lax.fori_loop(0, GB, _fire, 0)

        _stage_fire(0, 0)
        _stage_fire(1, 1)

        def _outer(i, _):
            g = i * 2
            for b in (0, 1):
                gg = g + b
                pltpu.make_async_copy(
                    hp.at[pl.ds(0, GB * 128)], rows[b], sems[b]).wait()

                def _scat(j, _):
                    pltpu.sync_copy(rows[b].at[pl.ds(j * 128, 128)],
                                    acc.at[dstv[b].at[j]], add=True)
                    return 0

                lax.fori_loop(0, GB, _scat, 0)

                @pl.when(gg + 2 < PROP_GROUPS)
                def _():
                    _stage_fire(gg + 2, b)

            return 0

        lax.fori_loop(0, PROP_GROUPS // 2, _outer, 0)
        plsc.subcore_barrier()
        pltpu.sync_copy(acc.at[pl.ds(nbase, TILE_N)],
                        out.at[pl.ds(nbase, TILE_N)])
        plsc.subcore_barrier()

    for p in range(2):
        for half in range(2):
            q = half * 2 + p

            @pl.when(c == half)
            def _(q=q):
                _do_pass(hps[q], outs[q])


_prop_call = pl.kernel(
    _prop_body,
    out_type=tuple(jax.ShapeDtypeStruct((NP, CW), F32) for _ in range(CHUNKS)),
    mesh=_MESH,
    scratch_types=[
        pltpu.VMEM((GB, 128), jnp.int32),
        pltpu.VMEM((GB, 128), jnp.int32),
        pltpu.VMEM((GB, 128), jnp.int32),
        pltpu.VMEM((GB, 128), jnp.int32),
        pltpu.VMEM((GB * 128, CW), F32),
        pltpu.VMEM((GB * 128, CW), F32),
        pltpu.VMEM((TILE_N // 8, CW), F32),
        pltpu.VMEM_SHARED((NP, CW), F32),
        pltpu.SemaphoreType.DMA,
        pltpu.SemaphoreType.DMA,
    ],
    compiler_params=_SC_PARAMS,
)


def _pre_body(x_ref, d0_ref, d1_ref, w1_ref, dinv_ref, *hp_refs):
    deg = d0_ref[...] + d1_ref[...] + 1.0
    dinv = lax.rsqrt(deg)
    dinv_ref[...] = dinv
    h = jnp.dot(x_ref[...], w1_ref[...], preferred_element_type=F32)
    for q in range(CHUNKS):
        hp_refs[q][...] = dinv * h[:, q * CW:(q + 1) * CW]


def _mid_body(a0, a1, a2, a3, h0, h1, h2, h3, dinv_ref, b1_ref, w2_ref,
              *out_refs):
    aggs = (a0, a1, a2, a3)
    hps = (h0, h1, h2, h3)
    dinv = dinv_ref[...]
    b = b1_ref[...]
    z = jnp.concatenate(
        [jax.nn.relu(dinv * (aggs[q][...] + hps[q][...])
                     + b[:, q * CW:(q + 1) * CW])
         for q in range(CHUNKS)], axis=1)
    h = jnp.dot(z, w2_ref[...], preferred_element_type=F32)
    for q in range(CHUNKS):
        out_refs[q][...] = dinv * h[:, q * CW:(q + 1) * CW]


def _post_body(a0, a1, a2, a3, h0, h1, h2, h3, dinv_ref, b2_ref, wl_ref,
               bl_ref, out_ref):
    aggs = (a0, a1, a2, a3)
    hps = (h0, h1, h2, h3)
    dinv = dinv_ref[...]
    b = b2_ref[...]
    z = jnp.concatenate(
        [jax.nn.relu(dinv * (aggs[q][...] + hps[q][...])
                     + b[:, q * CW:(q + 1) * CW])
         for q in range(CHUNKS)], axis=1)
    out_ref[...] = (jnp.dot(z, wl_ref[...], preferred_element_type=F32)
                    + bl_ref[...])


def _row_spec(w):
    return pl.BlockSpec((BN, w), lambda i: (i, 0))


def _full_spec(r, cdim):
    return pl.BlockSpec((r, cdim), lambda i: (0, 0))


_pre_call = pl.pallas_call(
    _pre_body,
    grid=(GRID,),
    in_specs=[_row_spec(F_IN), _row_spec(1), _row_spec(1),
              _full_spec(F_IN, F_H)],
    out_specs=[_row_spec(1)] + [_row_spec(CW)] * CHUNKS,
    out_shape=[jax.ShapeDtypeStruct((NP, 1), F32)]
    + [jax.ShapeDtypeStruct((NP, CW), F32) for _ in range(CHUNKS)],
)

_mid_call = pl.pallas_call(
    _mid_body,
    grid=(GRID,),
    in_specs=[_row_spec(CW)] * (2 * CHUNKS)
    + [_row_spec(1), _full_spec(1, F_H), _full_spec(F_H, F_H)],
    out_specs=[_row_spec(CW)] * CHUNKS,
    out_shape=[jax.ShapeDtypeStruct((NP, CW), F32) for _ in range(CHUNKS)],
)

_post_call = pl.pallas_call(
    _post_body,
    grid=(GRID,),
    in_specs=[_row_spec(CW)] * (2 * CHUNKS)
    + [_row_spec(1), _full_spec(1, F_H), _full_spec(F_H, F_OUT),
       _full_spec(1, F_OUT)],
    out_specs=_row_spec(F_OUT),
    out_shape=jax.ShapeDtypeStruct((NP, F_OUT), F32),
)


def kernel(x, edge_index, W1, b1, W2, b2, Wl, bl):
    src = edge_index[0]
    dst = edge_index[1]
    n_pad_rows = NP - N_NODES
    pad_idx = N_NODES + (jnp.arange(EP - N_EDGES, dtype=jnp.int32)
                         % n_pad_rows)
    src_p = jnp.concatenate([src, pad_idx]).reshape(ROWS, 128)
    dst_p = jnp.concatenate([dst, pad_idx]).reshape(ROWS, 128)
    x_p = jnp.pad(x, ((0, NP - N_NODES), (0, 0)))

    deg0, deg1 = _deg_call(dst_p)
    dinv, *h1p = _pre_call(x_p, deg0.reshape(NP, 1), deg1.reshape(NP, 1), W1)
    agg1 = _prop_call(*h1p, src_p, dst_p)
    h2p = _mid_call(*agg1, *h1p, dinv, b1.reshape(1, F_H), W2)
    agg2 = _prop_call(*h2p, src_p, dst_p)
    out = _post_call(*agg2, *h2p, dinv, b2.reshape(1, F_H), Wl,
                     bl.reshape(1, F_OUT))
    return out[:N_NODES]
